# baseline (speedup 1.0000x reference)
import functools

import numpy as np
import jax
import jax.numpy as jnp
from jax import lax
from jax.experimental import pallas as pl
from jax.experimental.pallas import tpu as pltpu

PS = 8192
D_IN = 1024
D_EMB = 512
N_CLS = 2
K_SEL = 2048
LEN_KEEP = PS - K_SEL


@functools.lru_cache(maxsize=1)
def _mask_consts():
    with jax.ensure_compile_time_eval():
        perm = np.asarray(jax.random.permutation(jax.random.key(42), PS))[:K_SEL]
    p_mask = np.zeros((PS,), np.int32)
    p_mask[perm] = 1
    pos = np.zeros((PS,), np.int32)
    pos[perm] = np.arange(K_SEL, dtype=np.int32)
    return p_mask, pos


_BI = 256
_BJ = 2048


def _ranks_body(acol_ref, arow_ref, out_ref):
    i = pl.program_id(0)
    j = pl.program_id(1)
    ai = acol_ref[...]
    aj = arow_ref[...]
    ii = i * _BI + lax.broadcasted_iota(jnp.int32, (_BI, 1), 0)
    jj = j * _BJ + lax.broadcasted_iota(jnp.int32, (1, _BJ), 1)
    lt = aj < ai
    tie = (aj == ai) & (jj < ii)
    cnt = jnp.sum((lt | tie).astype(jnp.int32), axis=1, keepdims=True)

    @pl.when(j == 0)
    def _():
        out_ref[...] = jnp.zeros_like(out_ref)

    out_ref[...] += cnt


def _ranks(attn_flat):
    a_col = attn_flat.reshape(PS, 1)
    a_row = attn_flat.reshape(1, PS)
    return pl.pallas_call(
        _ranks_body,
        grid=(PS // _BI, PS // _BJ),
        in_specs=[
            pl.BlockSpec((_BI, 1), lambda i, j: (i, 0)),
            pl.BlockSpec((1, _BJ), lambda i, j: (0, j)),
        ],
        out_specs=pl.BlockSpec((_BI, 1), lambda i, j: (i, 0)),
        out_shape=jax.ShapeDtypeStruct((PS, 1), jnp.int32),
    )(a_col, a_row)


_BM = 1024


def _emb_body(x_ref, w_ref, b_ref, emb_ref, sum_ref):
    xb = x_ref[...]
    w = w_ref[...]
    e = lax.dot_general(xb, w, (((1,), (1,)), ((), ())),
                        preferred_element_type=jnp.float32)
    e = jnp.maximum(e + b_ref[...], 0.0)
    emb_ref[...] = e

    @pl.when(pl.program_id(0) == 0)
    def _():
        sum_ref[...] = jnp.zeros_like(sum_ref)

    sum_ref[...] += jnp.sum(e, axis=0, keepdims=True)


def _embed(x_flat, W_emb, b_emb):
    return pl.pallas_call(
        _emb_body,
        grid=(PS // _BM,),
        in_specs=[
            pl.BlockSpec((_BM, D_IN), lambda i: (i, 0)),
            pl.BlockSpec((D_EMB, D_IN), lambda i: (0, 0)),
            pl.BlockSpec((1, D_EMB), lambda i: (0, 0)),
        ],
        out_specs=[
            pl.BlockSpec((_BM, D_EMB), lambda i: (i, 0)),
            pl.BlockSpec((1, D_EMB), lambda i: (0, 0)),
        ],
        out_shape=[
            jax.ShapeDtypeStruct((PS, D_EMB), jnp.float32),
            jax.ShapeDtypeStruct((1, D_EMB), jnp.float32),
        ],
    )(x_flat, W_emb, b_emb.reshape(1, D_EMB))


def kernel(x, attn, target, W_emb, b_emb, W_pred, b_pred):
    p_mask_np, pos_np = _mask_consts()
    p_mask = jnp.asarray(p_mask_np)
    pos = jnp.asarray(pos_np)

    ranks = _ranks(attn[0]).reshape(PS)
    emb, sum_all = _embed(x[0], W_emb, b_emb)

    m = p_mask[ranks]
    keep = 1 - m
    c_excl = jnp.cumsum(keep) - keep
    slot = jnp.where(m == 1, LEN_KEEP + pos[ranks], c_excl)
    ids = jnp.arange(PS, dtype=jnp.int32)
    mask_ids = jnp.zeros((PS,), jnp.int32).at[slot].set(ids)

    sum_masked = jnp.sum(emb[mask_ids[LEN_KEEP:]], axis=0)
    cls_feat = (sum_all[0] - sum_masked) / LEN_KEEP

    logits = (cls_feat @ W_pred.T + b_pred).reshape(1, N_CLS)
    loss = jnp.mean(jnp.sum(
        -jax.nn.softmax(target, axis=-1) * jax.nn.log_softmax(logits, axis=-1),
        axis=-1))
    return logits, loss, mask_ids.reshape(1, PS)


# speedup vs baseline: 1.7444x; 1.7444x over previous
import base64
import functools
import zlib

import numpy as np
import jax
import jax.numpy as jnp
from jax import lax
from jax.experimental import pallas as pl
from jax.experimental.pallas import tpu as pltpu
from jax.experimental.pallas import tpu_sc as plsc

PS = 8192
D_IN = 1024
D_EMB = 512
N_CLS = 2
K_SEL = 2048
LEN_KEEP = PS - K_SEL


_PERM_BLOB = (
    "c-jTQ31CR&0|4-M-}m*s_r3SMCrpcx5X0Y<=GrjIIY%s+x$kRoCJ_>4b1mcuVKhgM7RotCq-kc#G56f(wEypyfktCLX%nT3{"
    "Kvs%l0rR&?*3O&CwFKt+I^y2v1O@|IGHF>SEx%dkN;Ci^4I(KxRH2A^*66^P_7g^r?HXYZ|qq0KHGzwgK^@oLUHs=X(|k%zo"
    "eQFD*icKBu+y&Q3>%e(JrWn`sl^hL?Tu`h7SrS*mr`Z+}|wYu7T6toy>Lb3U&>hm!^}h-"
    "xKdC#)*TNYkYb1uf9*o#D<IW+$!9RKz941qwX1fnz;*h;7Y+gK22SO%_S<qy|96{fOuuUY36ek#C!Nd-yt1uuD{XOusAmpOh"
    "7r_YrmfGyYNW(7*F#jiL>!la-d$O+lDud-"
    "<W<R^!vM!c#fGz#L|bX+x%rek!9_%>=gGJoGvkf?;Ix%Dx>LUTz62$T!mi=WtOXVwp?d5*(ut={zxP;Kl1J1Gi#pkqw<ZI4@"
    "a>X;4+)a80vsXj#6;l+lu0Z!(tnS@h#z$aGgHx;P`4{zfZE?nlW;=l4Hl>O~O2^u^rF<EG$CHsfx;9eg__>dLoYc1T{1kIDP"
    "dDex2Ul?ZzKaTF{5YLDmTvBY^OdyI9+X-;s)REZU@P#5=Lya|5g+YEe{MIm9Vm0{xg-"
    ";#_xk`OoylavP$X7|ZsQE`UU?oyj_%`uz;q-YY&c*1}Y~IrhHT6<DAW^P#XIs>5DZ9)z^|LS5>#aCV|HcQ@eNVlM{Gw-"
    "(~xK+5jPj5kkF$HQD-pwrDCs90>7GnbhhlA##f#BPIBtGCsh=uiKqTroeRPJ%tKx}F?;KortrTnv0C)@M86TfBdmtiZEA1cW"
    "q!e8?bgT+}YS9ki47$Z?=KSxG<4EfbMdJG$;4=hhH4!>gdb3ABuG8#$1=Vzic4!tvA<CXKo7q*!^@02oj$krf@pf3k+6XVC_"
    "AI9!HxMHPv+>@VOP*%n{O%emMf#r#6rP6^RVYqWe+{TJIHjict#y~9}X0^Xi#<4p2Lx>@!gQ5CWpX5yt$q!0Ad{lC=KJR{z4"
    "x{I^<LiRoEKD-8$Ae*U84k5b1e8-"
    "TAG*`Wk)e;X%=lNu3iB`gNl`XOn_*=Zq&XQ{occ^?N^8e*BWs;u8*L7^TP5xV|iY<+PahF69b19(RGUHe6d!iHhz-"
    "(pJ0#BGf^#kEyx;+~P3cbV{B>f{_!VgEb`&7McrOIveS6l)}<mXYz)T>~r-^cADr?NFbXD$*JP*;N;_5(CKN-"
    "$5UPaMVinw#Y}5NC0j%2dO|&bm246L?*zOq~w%@tH~^*oJxEt_&90&G?3HITaVxH_u>wjrPVkX|&lw8lbFWQ^-eN0oIG(rA-"
    "N&)4fo0K2`4r@5qm|o6HC`Q5mcysD<1j)*@8oC~t?KJAyKodYj%KBq`6uG@`q;+pO&`rXZN;Yz0frZ9vDCsMX{{lx-xLoAf4"
    "p6EI9K#p;G%=vSg!#y40pv)y_wCHgv1K~V$BoGex47KT34Mf+Wz5It1hCRS=6pk;18GdQr(D|{hrVSivIi+%a#&fu_F_#hJf"
    "4pMD?Ke0!V&572-"
    "urMI`EP57R0M~+N*nVo7H<_D?9rh3CujP2D4K~(HB$jH+n9Alu>8zG%mPj^#nJbe$Y@NL+oMJ7OMo{HkvOUw;$%Tw%cXD>Qn"
    "l@h2$=l8x_YrYY{?>>WM^lfuWIdo0_{r#&JWQG_5_TVDgK~iC47W)a!$fM7_SlTK?$D*q^+04NN55Ky>SLGXGTj$m7BN=dL9"
    "S44;_+N}xX76rou#UghC4y7MQ`UVx(ArXj*jjdN>qU>H!G+c*r|F|u?@K%eIA_Q4Tcwb3q`zYe5f4OOZ}F3U$oC3q@|*<=p^"
    ";InCi7u-%&eiGvr6;8TT!JovmumF*=&HgzV^&@u^)=+Gz9)uM&?D13LS@yhhuP4iR}&s{gS>$b0EB^P>8RSr-"
    ")9HA%<)$$c)=!DR4nu$n$X?Z9O&j+_-$;TOqi!CmJ)G?RM*-ektB-@61}9lhXeu--it?DKB2yQA@Bs#%>cbZN2Jj^SS$&!|#"
    "vW~2)B)ULuZB}H#!rlUb(Hd~AAB3_^d^7WOkCEaWt9k%Ps#r_mDLSIO{;T=+CvZI)0l)G8*2;6`-"
    "mQBKty<jt`a<#)_fsa4N255VzgCZ041)o?AxyDjI^JnZXGle3ZT|$>Y<Qu|sQa&ihii2B9pYUjKG;9?7q7)O?<@)?Y`k{UY)"
    "%RccSu)OMYQMnhOpfF-HTW6!1+@}?&v^hA<7bSMfvpsX=LjQeM~x50Xo-3eSDW~p8EMu-ZNR7Ew5SE#gD~rOaE+Md#CZ}PPc"
    "{)ogLv~KdLR#xLv0fr7NoEdEEAe?U%MBS3ege$1ec=P<Wnp|YN@Ogim)w~#Xc~Ghna*+431W_HH~Z{iRh<vU{_MMzm9yD&Gt"
    "8|yGh9Wr4O{zu#Rjp{?yIo+6hmsm7oVPTxcXUWviGe(L4G&ewe+HOYonON7;*FL#8@Bt*6r)<Qq)DR2C{xXM(2&2b#ib=KH9"
    "S<em2BTFYT4St*7N6{^Hqpp&kSH`RxPqj+0g<JYHZ_(zo;N`vsIIF5an7^knJ21fJE8e%W2AzDWIZZiEqtE;@0;)tro3~jLR"
    "rrgK4B#l?foD=#!vy<B2xoOr!Rq>C4Da<M62N}}M;SEfezcX&5fc-"
    "<dtpCUUi1qTSE57C{ou!`oC|4mr!V~aecy~p$^87&TMikI1Ro0qHR-z|j^@Y32h~U$(2R{YZgxX+}xEXPD^C-"
    "zE3BN_X%@ptfKM<^DEO^={V0QQqt2wIwiE><hLZ4Bt;n$cMY!_vrwTg~)zcxzgwR|%l2eIZeKyU@<ZRFzb_()!*JeH0GudKk"
    "Ez>l(b+JJlCl>9GYDc_wcmTQY|(SNa9<VO4w=65O)15pijjlJ4wK;IJR07l2Kx1GJz<4_|G!Znzs24N+<xn0dILD^gy`jENG"
    "9hKU9tNeVF0|!uhRo?9D?eUKpqs16FKN3}3ZXe9yQ}Kmb71ecj6JIHHqm^NQyfgh++)NCUd2)oD6`3dz*3r9qXR$mwje23Pw"
    "-cyT^|`v-{6waSnS2s<(B5KfR^GsKz;yXzr9QS7>+a-"
    "fo}cb6G(?YuY32&=a!|u6P<w+|`=olF{LGk+&C!N>&D5t(Lp_#h1yrUfHAy-EPm8k=GT#?+gHNgZxF+4BRrQ6i6b?b2aoa!V"
    "?!k_MQJ@if!Cfi8XI&r!qKBCk7D?Ud3gp-"
    "@)@bW&q1x#X>8Lpxsn0@F;Y+?^bk2z(XLGFC*gYpKMk9@6X}7vwk8{p~HDniW6?Y}L7<~t-"
    "MZ<$+V`7*sK=~`XCR<&k__Z4JDnd>kM5yGf@QFDKdlQ!M1apGz<@yk6c9cMTq;8_0L<vX?1?dz!QD`O(AxqVhQd4kAoJujw8"
    "2l-"
    "*K<MW_$A>UqdbQYcJu@7S#i=ixJo9I&xsa=E58KnJbT*nVTER#r8U8NS4~F~a#SS)M5>`iHhdRpo(5@yP43FZL_cxblK7<De"
    "-Q8qAg4My#QX7A+IUUs{S*2Cjnv7u#vMUu~x5AP1A3+lJSjceRF(;~Vbb|6qxW*ngI_X=(s-WE1LymQ-hiBE*e*XxOLjlILg"
    "<XjW)KBmMofBD17k!BLr!$XTVeX)S?0Ph=hf9Tj;54m0*&^8GY}7jI@8b286ZAoI7dyad!7ukG;`!KJeJ)p30bWz(p*_%iN#"
    "wZyBj=GPJdSVW{=%e1e=D3YJG>}NG48st#9C%+bc{|C%gEEgIW9}9sSZ?E(1(OedakvEodsL*jg&lh2RIiVAzAu8XR+mR9re"
    "1b#Er8g@>iOap2MHj^>iR*%O&<{R`PP43}u;gSlvb?=>63*(RQh(@SnIo(1q7t9iftNR(Ko@Qm>o8nPq+tXOq$*+@|)mR-yI"
    "MU~Gnd7$#W1a=qbrBR$NB(s`R2%pDVUx+@ve9Ve1z0a=m#Pio7(=_CYmqj;DqUF2}d3yXOT45O~1i*~wjz&xSEa>Z<nl*Rr5"
    "$5{WM%R&z$Mr@1eTDpC}Ju9CeEGpjk+OHIpdrxg5%ySaMzRDfDF>k^`YPMWU-"
    "l=~l{S<c6N|_9;P;4z{@fp}_>|c<hWVjpnSKt6R$^8<J3ip8?l&m+hw?)f|{lFA&_#N%x&OK@&_XW*Mcf|+@@C~_vInTIZl&"
    "Qm1)5{emp~hG){Tl7DMtU2Shp0%I9&LwB^tZ%zQ9Rb$TY=UDv*8=y9`=V(6-"
    "p>ig4j#FkJZ8~AX5DAg(MxY4Zr}sP|tBLqHbO;ND7uK6up(-0!G>f-"
    "q|P&?@2tS(p&Ty#9?cM*2>K1Cs<>It;!^MQkW|@B~H?3P^o13tuad9jaq9J@wRdatYR$?`v=K<8SF#tu_5=8oEFq$e-"
    "C2J80vqhl|47uP7DBbg?VA+AXA(#Rp8GO=~0ne6K@mc;N6rS;#FZ6>PB^dy~0}VP_5LfMVwOs%(QNcOO#xF3*FT`WzTW8h#1"
    "-w+`z|xA<;!8O?pAtY@t7t%M7LpHG?K<ihbY7!RGm_+(Rp4TZ(@&qJ5b;t!#G_-"
    "2#0Ohne5mS>%ea7<(R`XU`LLf__+KeYx|E_ao}1S2A`h6Ik2prxxoq^jGv}&Plem_LiRz4fA)>1B}I37ZAA_<P$>Iw{UGi9b"
    "!CQ$xbKFXm?S6Z?n`<+9&UlL@NWH;7hcn;yAm&xP-"
    "c)0=ccthr77f(LT!no85ImNPQ3XLr!ZSSzH_whs^~4=?tBuntq8=*BF6(e}u4CD+2ZW964WYAgooI$Uj8YK#o>!<x(xRN#sj"
    "(i;7Y5_;}$RtSdf;ka0<$FKBAIcpiwx{{h0%Z=n"
)


@functools.lru_cache(maxsize=1)
def _mask_consts():
    perm = np.frombuffer(zlib.decompress(base64.b85decode(_PERM_BLOB)),
                         dtype=np.int16).astype(np.int32)
    p_mask = np.zeros((PS,), np.int32)
    p_mask[perm] = 1
    pos = np.zeros((PS,), np.int32)
    pos[perm] = np.arange(K_SEL, dtype=np.int32)
    return p_mask, pos


_B = 1024


def _fused_body(acol_ref, arow_ref, x_ref, w_ref, b_ref,
                ranks_ref, emb_ref, sum_ref, acc_ref):
    i = pl.program_id(0)
    j = pl.program_id(1)

    @pl.when(j == 0)
    def _():
        e = lax.dot_general(x_ref[...], w_ref[...], (((1,), (1,)), ((), ())),
                            preferred_element_type=jnp.float32)
        e = jnp.maximum(e + b_ref[...], 0.0)
        emb_ref[...] = e

        @pl.when(i == 0)
        def _():
            sum_ref[...] = jnp.zeros_like(sum_ref)

        sum_ref[...] += jnp.sum(e, axis=0, keepdims=True)
        acc_ref[...] = jnp.zeros_like(acc_ref)

    ai = acol_ref[...]
    aj = arow_ref[...]

    def _acc(cmp):
        c = cmp.astype(jnp.int32)
        acc_ref[...] += ((c[:, 0:128] + c[:, 128:256])
                         + (c[:, 256:384] + c[:, 384:512])
                         + (c[:, 512:640] + c[:, 640:768])
                         + (c[:, 768:896] + c[:, 896:1024]))

    @pl.when(j < i)
    def _():
        _acc(aj <= ai)

    @pl.when(j > i)
    def _():
        _acc(aj < ai)

    @pl.when(j == i)
    def _():
        ii = i * _B + lax.broadcasted_iota(jnp.int32, (_B, 1), 0)
        jj = j * _B + lax.broadcasted_iota(jnp.int32, (1, _B), 1)
        _acc((aj < ai) | ((aj == ai) & (jj < ii)))

    @pl.when(j == pl.num_programs(1) - 1)
    def _():
        ranks_ref[...] = jnp.sum(acc_ref[...], axis=1, keepdims=True)


def _ranks_and_embed(attn_flat, x_flat, W_emb, b_emb):
    a_col = attn_flat.reshape(PS, 1)
    a_row = attn_flat.reshape(1, PS)
    return pl.pallas_call(
        _fused_body,
        grid=(PS // _B, PS // _B),
        in_specs=[
            pl.BlockSpec((_B, 1), lambda i, j: (i, 0)),
            pl.BlockSpec((1, _B), lambda i, j: (0, j)),
            pl.BlockSpec((_B, D_IN), lambda i, j: (i, 0)),
            pl.BlockSpec((D_EMB, D_IN), lambda i, j: (0, 0)),
            pl.BlockSpec((1, D_EMB), lambda i, j: (0, 0)),
        ],
        out_specs=[
            pl.BlockSpec((_B, 1), lambda i, j: (i, 0)),
            pl.BlockSpec((_B, D_EMB), lambda i, j: (i, 0)),
            pl.BlockSpec((1, D_EMB), lambda i, j: (0, 0)),
        ],
        out_shape=[
            jax.ShapeDtypeStruct((PS, 1), jnp.int32),
            jax.ShapeDtypeStruct((PS, D_EMB), jnp.float32),
            jax.ShapeDtypeStruct((1, D_EMB), jnp.float32),
        ],
        scratch_shapes=[pltpu.VMEM((_B, 128), jnp.int32)],
    )(a_col, a_row, x_flat, W_emb, b_emb.reshape(1, D_EMB))


_NS = 16
_KW = K_SEL // _NS
_TW = PS // _NS
_VW = _TW // 16


def _sc_mask_and_reduce(ranks, p_mask, pos, emb):
    mesh = plsc.VectorSubcoreMesh(core_axis_name="c", subcore_axis_name="s",
                                  num_cores=1, num_subcores=_NS)

    @functools.partial(
        pl.kernel,
        out_type=[
            jax.ShapeDtypeStruct((PS,), jnp.int32),
            jax.ShapeDtypeStruct((_NS, D_EMB), jnp.float32),
        ],
        mesh=mesh,
        compiler_params=pltpu.CompilerParams(needs_layout_passes=False),
        scratch_types=[
            pltpu.VMEM((_TW,), jnp.int32),
            pltpu.VMEM((PS,), jnp.int32),
            pltpu.VMEM((PS,), jnp.int32),
            pltpu.VMEM((_TW,), jnp.int32),
            pltpu.VMEM((_TW,), jnp.int32),
            pltpu.VMEM((16,), jnp.int32),
            pltpu.VMEM((_NS, 16), jnp.int32),
            pltpu.VMEM((PS,), jnp.int32),
            pltpu.VMEM((PS,), jnp.int32),
            pltpu.VMEM_SHARED((_NS, 16), jnp.int32),
            pltpu.VMEM_SHARED((PS,), jnp.int32),
            pltpu.VMEM_SHARED((K_SEL,), jnp.int32),
            pltpu.VMEM((_KW,), jnp.int32),
            pltpu.VMEM((_KW, D_EMB), jnp.float32),
            pltpu.VMEM((D_EMB,), jnp.float32),
            pltpu.SemaphoreType.DMA,
        ],
    )
    def k(ranks_hbm, pmask_hbm, pos_hbm, emb_hbm, ids_hbm, part_hbm,
          ranks_v, pmask_v, pos_v, keep_v, slot_loc, cnt_v, tot_v,
          slots_v, out_v, tot_sh, slots_sh, masked_sh,
          idx_v, rows_v, part_v, sem):
        wid = lax.axis_index("s")
        tbase = wid * _TW

        pltpu.sync_copy(ranks_hbm.at[pl.ds(pl.multiple_of(tbase, 16), _TW)],
                        ranks_v)
        pltpu.sync_copy(pmask_hbm, pmask_v)
        pltpu.sync_copy(pos_hbm, pos_v)
        acc = jnp.zeros((16,), jnp.int32)
        for v in range(_VW):
            r = ranks_v[pl.ds(v * 16, 16)]
            keep = 1 - plsc.load_gather(pmask_v, [r])
            keep_v[pl.ds(v * 16, 16)] = keep
            acc = acc + keep
        cnt_v[...] = acc
        pltpu.sync_copy(cnt_v, tot_sh.at[wid])
        plsc.subcore_barrier()

        pltpu.sync_copy(tot_sh, tot_v)
        psum = jnp.zeros((16,), jnp.int32)
        for kk in range(_NS):
            row = tot_v[kk, pl.ds(0, 16)]
            psum = psum + jnp.where(kk < wid, row, jnp.zeros((16,), jnp.int32))
        carry = jnp.sum(psum)
        for v in range(_VW):
            sl = pl.ds(v * 16, 16)
            r = ranks_v[sl]
            keep = keep_v[sl]
            cs = plsc.cumsum(keep)
            kept_slot = carry + cs - keep
            p = plsc.load_gather(pos_v, [r])
            slot_loc[sl] = jnp.where(keep == 0, LEN_KEEP + p, kept_slot)
            carry = carry + jnp.sum(keep)
        pltpu.sync_copy(slot_loc,
                        slots_sh.at[pl.ds(pl.multiple_of(tbase, 16), _TW)])
        plsc.subcore_barrier()

        @pl.when(wid == 0)
        def _scatter():
            pltpu.sync_copy(slots_sh, slots_v)

            def body(g, _):
                for u in range(8):
                    v = g * 8 + u
                    sl = pl.ds(pl.multiple_of(v * 16, 16), 16)
                    slot = slots_v[sl]
                    ids = v * 16 + lax.iota(jnp.int32, 16)
                    plsc.store_scatter(out_v, [slot], ids)
                return 0

            lax.fori_loop(0, PS // 128, body, jnp.int32(0))
            pltpu.sync_copy(out_v, ids_hbm)
            pltpu.sync_copy(out_v.at[pl.ds(LEN_KEEP, K_SEL)], masked_sh)

        plsc.subcore_barrier()

        pltpu.sync_copy(masked_sh.at[pl.ds(pl.multiple_of(wid * _KW, 16),
                                           _KW)], idx_v)
        pltpu.async_copy(emb_hbm.at[idx_v], rows_v, sem).wait()

        def rbody(kk, accs):
            return tuple(
                accs[c] + rows_v[kk, pl.ds(c * 16, 16)]
                for c in range(D_EMB // 16))

        accs = lax.fori_loop(
            0, _KW, rbody,
            tuple(jnp.zeros((16,), jnp.float32) for _ in range(D_EMB // 16)))
        for c in range(D_EMB // 16):
            part_v[pl.ds(c * 16, 16)] = accs[c]
        pltpu.sync_copy(part_v, part_hbm.at[wid])

    return k(ranks, p_mask, pos, emb)


def _combine_body(sum_ref, part_ref, wp_ref, bp_ref, tgt_ref,
                  logit_ref, loss_ref):
    s = sum_ref[...]
    p = jnp.sum(part_ref[...], axis=0, keepdims=True)
    cls = (s - p) * (1.0 / LEN_KEEP)
    logits = lax.dot_general(cls, wp_ref[...], (((1,), (1,)), ((), ())),
                             preferred_element_type=jnp.float32) + bp_ref[...]
    logit_ref[...] = logits
    t = tgt_ref[...]
    te = jnp.exp(t - jnp.max(t, axis=-1, keepdims=True))
    sm_t = te / jnp.sum(te, axis=-1, keepdims=True)
    le = logits - jnp.max(logits, axis=-1, keepdims=True)
    logsm = le - jnp.log(jnp.sum(jnp.exp(le), axis=-1, keepdims=True))
    loss_ref[...] = jnp.sum(-sm_t * logsm, axis=-1, keepdims=True)


def _combine(sum_all, partials, W_pred, b_pred, target):
    return pl.pallas_call(
        _combine_body,
        out_shape=[
            jax.ShapeDtypeStruct((1, N_CLS), jnp.float32),
            jax.ShapeDtypeStruct((1, 1), jnp.float32),
        ],
    )(sum_all, partials, W_pred, b_pred.reshape(1, N_CLS), target)


def kernel(x, attn, target, W_emb, b_emb, W_pred, b_pred):
    p_mask_np, pos_np = _mask_consts()
    p_mask = jnp.asarray(p_mask_np)
    pos = jnp.asarray(pos_np)

    ranks_col, emb, sum_all = _ranks_and_embed(attn[0], x[0], W_emb, b_emb)
    ranks = ranks_col.reshape(PS)

    mask_ids, partials = _sc_mask_and_reduce(ranks, p_mask, pos, emb)
    logits, loss = _combine(sum_all, partials, W_pred, b_pred, target)
    return logits, loss.reshape(()), mask_ids.reshape(1, PS)
